# baseline (speedup 1.0000x reference)
import functools

import jax
import jax.numpy as jnp
from jax import lax
from jax.experimental import pallas as pl
from jax.experimental.pallas import tpu as pltpu
from jax.experimental.pallas import tpu_sc as plsc

N = 16384
COLS = 64
HCOLS = COLS // 2
NC = 2
NS = 16
K = 512
NBUF = 3
ROWS_PER_TILE = N // NS


def _spmm2_kernel(nnz_pad):
    e_tile = nnz_pad // NS
    n_chunks = e_tile // K
    assert n_chunks % NBUF == 0
    mesh = plsc.VectorSubcoreMesh(
        core_axis_name="c", subcore_axis_name="s",
        num_cores=NC, num_subcores=NS)

    @functools.partial(
        pl.kernel,
        out_type=jax.ShapeDtypeStruct((N, COLS), jnp.float32),
        mesh=mesh,
        compiler_params=pltpu.CompilerParams(use_tc_tiling_on_sc=False),
        scratch_types=(
            [pltpu.VMEM_SHARED((N, HCOLS), jnp.float32)]
            + [pltpu.VMEM_SHARED((N, HCOLS), jnp.float32)]
            + [pltpu.VMEM((K,), jnp.int32)] * NBUF
            + [pltpu.VMEM((K,), jnp.int32)] * NBUF
            + [pltpu.VMEM((K,), jnp.float32)] * NBUF
            + [pltpu.VMEM((K, HCOLS), jnp.float32)] * NBUF
            + [pltpu.SemaphoreType.DMA] * (3 * NBUF)
        ),
    )
    def k(xh, coli, rowi, wts, zrows, out, acc, x1s, *bufs):
        cb = bufs[0:NBUF]
        rb = bufs[NBUF:2 * NBUF]
        wv = bufs[2 * NBUF:3 * NBUF]
        gv = bufs[3 * NBUF:4 * NBUF]
        isem = bufs[4 * NBUF:5 * NBUF]
        gsem = bufs[5 * NBUF:6 * NBUF]
        ssem = bufs[6 * NBUF:7 * NBUF]
        h = lax.axis_index("c")
        sid = lax.axis_index("s")
        row_base = sid * ROWS_PER_TILE
        e_base = sid * e_tile

        def col_copy(c, j):
            return pltpu.make_async_copy(
                coli.at[pl.ds(e_base + c * K, K)], cb[j], isem[j])

        def row_copy(c, j):
            return pltpu.make_async_copy(
                rowi.at[pl.ds(e_base + c * K, K)], rb[j], isem[j])

        def w_copy(c, j):
            return pltpu.make_async_copy(
                wts.at[pl.ds(e_base + c * K, K)], wv[j], isem[j])

        def start_idx(c, j):
            col_copy(c, j).start()
            row_copy(c, j).start()
            w_copy(c, j).start()

        def wait_idx(c, j):
            col_copy(c, j).wait()
            row_copy(c, j).wait()
            w_copy(c, j).wait()

        def col_offset(j, hoff):
            def off_body(g, _):
                base = g * 64
                for r in range(4):
                    s = pl.ds(base + r * 16, 16)
                    cb[j][s] = cb[j][s] + hoff
                return 0

            lax.fori_loop(0, K // 64, off_body, 0, unroll=False)

        def gather_copy(src_hbm, j):
            return pltpu.make_async_copy(src_hbm.at[cb[j]], gv[j], gsem[j])

        def scatter_copy(dst, j):
            return pltpu.make_async_copy(gv[j], dst.at[rb[j]], ssem[j])

        def scale(j):
            def scale_body(g, _):
                base = g * 16
                w16 = wv[j][pl.ds(base, 16)]
                for r in range(16):
                    i = base + r
                    w = w16[r]
                    gv[j][i, pl.ds(0, 16)] = gv[j][i, pl.ds(0, 16)] * w
                    gv[j][i, pl.ds(16, 16)] = gv[j][i, pl.ds(16, 16)] * w
                return 0

            lax.fori_loop(0, K // 16, scale_body, 0, unroll=False)

        def edge_loop(src, dst, hoff):
            start_idx(0, 0)
            start_idx(1, 1)
            wait_idx(0, 0)
            if hoff is not None:
                col_offset(0, hoff)
            gather_copy(src, 0).start()

            def outer_body(t, _):
                for j in range(NBUF):
                    c = NBUF * t + j
                    jn = (j + 1) % NBUF
                    jp = (j + 2) % NBUF
                    @pl.when(c + 1 < n_chunks)
                    def _():
                        wait_idx(c + 1, jn)
                        if hoff is not None:
                            col_offset(jn, hoff)
                        gather_copy(src, jn).start()
                    gather_copy(src, j).wait()
                    scale(j)
                    scatter_copy(dst, j).start(add=True)
                    @pl.when((c + 2 < n_chunks) & (c >= 1))
                    def _():
                        scatter_copy(dst, jp).wait()
                    @pl.when(c + 2 < n_chunks)
                    def _():
                        start_idx(c + 2, jp)
                return 0

            lax.fori_loop(0, n_chunks // NBUF, outer_body, 0, unroll=False)
            for j in range(NBUF):
                scatter_copy(dst, j).wait()

        pltpu.sync_copy(zrows, acc.at[pl.ds(row_base, ROWS_PER_TILE)])
        pltpu.sync_copy(zrows, x1s.at[pl.ds(row_base, ROWS_PER_TILE)])
        plsc.subcore_barrier()
        edge_loop(xh, x1s, h * N)
        plsc.subcore_barrier()
        edge_loop(x1s, acc, None)
        plsc.subcore_barrier()
        pltpu.sync_copy(
            acc.at[pl.ds(row_base, ROWS_PER_TILE)],
            out.at[pl.ds(row_base, ROWS_PER_TILE), pl.ds(h * HCOLS, HCOLS)])

    return k


def kernel(x, indices, weights):
    nnz = weights.shape[0]
    chunk_all = NS * K * NBUF
    nnz_pad = ((nnz + chunk_all - 1) // chunk_all) * chunk_all
    pad = nnz_pad - nnz

    row = indices[0]
    col = indices[1]
    if pad:
        row = jnp.pad(row, (0, pad))
        col = jnp.pad(col, (0, pad))
        weights = jnp.pad(weights, (0, pad))
    xh = jnp.concatenate([x[:, :HCOLS], x[:, HCOLS:]], axis=0)
    zrows = jnp.zeros((ROWS_PER_TILE, HCOLS), jnp.float32)

    return _spmm2_kernel(nnz_pad)(xh, col, row, weights, zrows)


# speedup vs baseline: 1.2767x; 1.2767x over previous
import functools

import jax
import jax.numpy as jnp
from jax import lax
from jax.experimental import pallas as pl
from jax.experimental.pallas import tpu as pltpu
from jax.experimental.pallas import tpu_sc as plsc

N = 16384
COLS = 64
HCOLS = COLS // 2
NC = 2
NS = 16
K = 768
NBUF = 3
ROWS_PER_TILE = N // NS


def _spmm2_kernel(nnz):
    e_tile = ((nnz + NS - 1) // NS + 7) // 8 * 8
    n_chunks = (e_tile + K - 1) // K
    m14 = n_chunks * K - e_tile
    m15 = (15 * e_tile + (n_chunks - 1) * K) - (nnz - K)
    assert n_chunks % NBUF == 0
    assert (n_chunks - 1) * K < e_tile <= n_chunks * K
    assert 15 * e_tile + (n_chunks - 1) * K < nnz
    assert 0 <= m14 and 0 <= m15 < K
    mesh = plsc.VectorSubcoreMesh(
        core_axis_name="c", subcore_axis_name="s",
        num_cores=NC, num_subcores=NS)

    @functools.partial(
        pl.kernel,
        out_type=(
            jax.ShapeDtypeStruct((N, COLS), jnp.float32),
            jax.ShapeDtypeStruct((NC * N, HCOLS), jnp.float32),
        ),
        mesh=mesh,
        compiler_params=pltpu.CompilerParams(use_tc_tiling_on_sc=False),
        scratch_types=(
            [pltpu.VMEM_SHARED((N, HCOLS), jnp.float32)]
            + [pltpu.VMEM((K,), jnp.int32)] * NBUF
            + [pltpu.VMEM((K,), jnp.int32)] * NBUF
            + [pltpu.VMEM((K,), jnp.float32)] * NBUF
            + [pltpu.VMEM((K, HCOLS), jnp.float32)] * NBUF
            + [pltpu.SemaphoreType.DMA] * (3 * NBUF)
        ),
    )
    def k(xh, coli, rowi, wts, colt, rowt, wt, zrows, out, x1h, acc, *bufs):
        cb = bufs[0:NBUF]
        rb = bufs[NBUF:2 * NBUF]
        wv = bufs[2 * NBUF:3 * NBUF]
        gv = bufs[3 * NBUF:4 * NBUF]
        isem = bufs[4 * NBUF:5 * NBUF]
        gsem = bufs[5 * NBUF:6 * NBUF]
        ssem = bufs[6 * NBUF:7 * NBUF]
        h = lax.axis_index("c")
        sid = lax.axis_index("s")
        row_base = sid * ROWS_PER_TILE
        e_base = sid * e_tile

        def chunk_start(c):
            return e_base + jnp.minimum(c * K, e_tile - K)

        def col_copy(c, j):
            return pltpu.make_async_copy(
                coli.at[pl.ds(chunk_start(c), K)], cb[j], isem[j])

        def row_copy(c, j):
            return pltpu.make_async_copy(
                rowi.at[pl.ds(chunk_start(c), K)], rb[j], isem[j])

        def w_copy(c, j):
            return pltpu.make_async_copy(
                wts.at[pl.ds(chunk_start(c), K)], wv[j], isem[j])

        def start_idx(c, j):
            side = (c == n_chunks - 1) & (sid == NS - 1)

            @pl.when(side)
            def _():
                pltpu.make_async_copy(colt, cb[j], isem[j]).start()
                pltpu.make_async_copy(rowt, rb[j], isem[j]).start()
                pltpu.make_async_copy(wt, wv[j], isem[j]).start()

            @pl.when(jnp.logical_not(side))
            def _():
                col_copy(c, j).start()
                row_copy(c, j).start()
                w_copy(c, j).start()

        def wait_idx(c, j):
            pltpu.make_async_copy(colt, cb[j], isem[j]).wait()
            pltpu.make_async_copy(rowt, rb[j], isem[j]).wait()
            pltpu.make_async_copy(wt, wv[j], isem[j]).wait()

        def col_offset(j, hoff):
            def off_body(g, _):
                base = g * 64
                for r in range(4):
                    s = pl.ds(base + r * 16, 16)
                    cb[j][s] = cb[j][s] + hoff
                return 0

            lax.fori_loop(0, K // 64, off_body, 0, unroll=False)

        def gather_copy(src_hbm, j):
            return pltpu.make_async_copy(src_hbm.at[cb[j]], gv[j], gsem[j])

        def scatter_copy(j):
            return pltpu.make_async_copy(gv[j], acc.at[rb[j]], ssem[j])

        def scale(j, m):
            lane = lax.iota(jnp.int32, 16)

            def scale_body(g, _):
                base = g * 16
                w16 = wv[j][pl.ds(base, 16)]
                w16 = jnp.where(lane + base >= m, w16, 0.0)
                for r in range(16):
                    i = base + r
                    w = w16[r]
                    gv[j][i, pl.ds(0, 16)] = gv[j][i, pl.ds(0, 16)] * w
                    gv[j][i, pl.ds(16, 16)] = gv[j][i, pl.ds(16, 16)] * w
                return 0

            lax.fori_loop(0, K // 16, scale_body, 0, unroll=False)

        def edge_loop(src_hbm):
            hoff = h * N
            start_idx(0, 0)
            start_idx(1, 1)
            wait_idx(0, 0)
            col_offset(0, hoff)
            gather_copy(src_hbm, 0).start()

            def outer_body(t, _):
                for j in range(NBUF):
                    c = NBUF * t + j
                    jn = (j + 1) % NBUF
                    jp = (j + 2) % NBUF
                    @pl.when(c + 1 < n_chunks)
                    def _():
                        wait_idx(c + 1, jn)
                        col_offset(jn, hoff)
                        gather_copy(src_hbm, jn).start()
                    gather_copy(src_hbm, j).wait()
                    m = jnp.where(
                        c == n_chunks - 1,
                        jnp.where(sid == NS - 1, m15, m14), 0)
                    scale(j, m)
                    scatter_copy(j).start(add=True)
                    @pl.when((c + 2 < n_chunks) & (c >= 1))
                    def _():
                        scatter_copy(jp).wait()
                    @pl.when(c + 2 < n_chunks)
                    def _():
                        start_idx(c + 2, jp)
                return 0

            lax.fori_loop(0, n_chunks // NBUF, outer_body, 0, unroll=False)
            for j in range(NBUF):
                scatter_copy(j).wait()

        pltpu.sync_copy(zrows, acc.at[pl.ds(row_base, ROWS_PER_TILE)])
        plsc.subcore_barrier()
        edge_loop(xh)
        plsc.subcore_barrier()
        pltpu.sync_copy(
            acc.at[pl.ds(row_base, ROWS_PER_TILE)],
            x1h.at[pl.ds(h * N + row_base, ROWS_PER_TILE)])
        pltpu.sync_copy(zrows, acc.at[pl.ds(row_base, ROWS_PER_TILE)])
        plsc.subcore_barrier()
        edge_loop(x1h)
        plsc.subcore_barrier()
        pltpu.sync_copy(
            acc.at[pl.ds(row_base, ROWS_PER_TILE)],
            out.at[pl.ds(row_base, ROWS_PER_TILE), pl.ds(h * HCOLS, HCOLS)])

    return k


def kernel(x, indices, weights):
    nnz = weights.shape[0]
    row = indices[0]
    col = indices[1]
    colt = lax.slice(col, (nnz - K,), (nnz,))
    rowt = lax.slice(row, (nnz - K,), (nnz,))
    wt = lax.slice(weights, (nnz - K,), (nnz,))
    xh = jnp.concatenate([x[:, :HCOLS], x[:, HCOLS:]], axis=0)
    zrows = jnp.zeros((ROWS_PER_TILE, HCOLS), jnp.float32)

    out, _ = _spmm2_kernel(nnz)(
        xh, col, row, weights, colt, rowt, wt, zrows)
    return out
